# baseline (speedup 1.0000x reference)
import functools

import jax
import jax.numpy as jnp
from jax import lax
from jax.experimental import pallas as pl
from jax.experimental.pallas import tpu as pltpu
from jax.experimental.pallas import tpu_sc as plsc

N = 10000
E = 320000
D = 128
D_EDGE = 16
AVG_D_LOG = 3.5

NC = 2
NS = 16
L = 16
NW = NC * NS
NPART = 80
PART = 128
NPAD = NPART * PART
CHUNK = 1600
NCHUNK = E // CHUNK
WAVE = 64
DRAIN_AT = WAVE - 2 * L
R = D // L

_NEG = -1e30
_POS = 1e30


def _mm_kernel(x_ref, w_ref, b_ref, o_ref, *, relu):
    acc = jnp.dot(x_ref[...], w_ref[...], preferred_element_type=jnp.float32)
    acc = acc + b_ref[...]
    if relu:
        acc = jnp.maximum(acc, 0.0)
    o_ref[...] = acc


def _matmul(x, w, b, *, block_m, relu=False):
    m, k = x.shape
    _, n = w.shape
    return pl.pallas_call(
        functools.partial(_mm_kernel, relu=relu),
        grid=(m // block_m,),
        in_specs=[
            pl.BlockSpec((block_m, k), lambda i: (i, 0)),
            pl.BlockSpec((k, n), lambda i: (0, 0)),
            pl.BlockSpec((n,), lambda i: (0,)),
        ],
        out_specs=pl.BlockSpec((block_m, n), lambda i: (i, 0)),
        out_shape=jax.ShapeDtypeStruct((m, n), jnp.float32),
    )(x, w, b)


def _pre_kernel(h_ref, ws_ref, wd_ref, a_ref, b_ref):
    h = h_ref[...]
    a_ref[...] = jnp.dot(h, ws_ref[...], preferred_element_type=jnp.float32)
    b_ref[...] = jnp.dot(h, wd_ref[...], preferred_element_type=jnp.float32)


def _pre_matmuls(hp, w_s, w_d):
    bm = 1024
    return pl.pallas_call(
        _pre_kernel,
        grid=(NPAD // bm,),
        in_specs=[
            pl.BlockSpec((bm, D), lambda i: (i, 0)),
            pl.BlockSpec((D, D), lambda i: (0, 0)),
            pl.BlockSpec((D, D), lambda i: (0, 0)),
        ],
        out_specs=[
            pl.BlockSpec((bm, D), lambda i: (i, 0)),
            pl.BlockSpec((bm, D), lambda i: (i, 0)),
        ],
        out_shape=[
            jax.ShapeDtypeStruct((NPAD, D), jnp.float32),
            jax.ShapeDtypeStruct((NPAD, D), jnp.float32),
        ],
    )(hp, w_s, w_d)


def _sc_body(a_hbm, b_hbm, cm_hbm, src_hbm, dst_hbm,
             s_out, mx_out, mn_out, sq_out, deg_out,
             acc_s, acc_mx, acc_mn, acc_sq, degv, bpart,
             srcc0, dstc0, srcc1, dstc1, msrc, meid, mdst, arows, crows,
             sema, semc, semch0, semch1):
    wid = lax.axis_index("s") * NC + lax.axis_index("c")
    iota = lax.iota(jnp.int32, L)
    ones = jnp.ones((L,), jnp.float32)
    zero16i = jnp.zeros((L,), jnp.int32)

    e0 = jnp.where(iota == 0, 1.0, 0.0).astype(jnp.float32)

    def accumulate_wave(j, _):
        d = mdst[pl.ds(j, L)][0]
        base = d * D
        dg = pl.ds(d, L)
        degv[dg] = degv[dg] + e0
        for r in range(R):
            av = arows[j, pl.ds(r * L, L)]
            cv = crows[j, pl.ds(r * L, L)]
            bv = bpart[d, pl.ds(r * L, L)]
            m = jnp.maximum(av + cv + bv, 0.0)
            o = pl.ds(base + r * L, L)
            plsc.addupdate(acc_s.at[o], m)
            plsc.addupdate(acc_sq.at[o], m * m)
            acc_mx[o] = jnp.maximum(acc_mx[o], m)
            acc_mn[o] = jnp.minimum(acc_mn[o], m)
        return 0

    def drain(voffv):
        for w in range(WAVE // L):
            g = w * L + iota
            pm = g >= voffv
            o = pl.ds(w * L, L)
            msrc[o] = jnp.where(pm, zero16i, msrc[o])
            meid[o] = jnp.where(pm, zero16i, meid[o])
            mdst[o] = jnp.where(pm, jnp.full((L,), PART, jnp.int32), mdst[o])
        cpa = pltpu.async_copy(a_hbm.at[msrc], arows, sema)
        cpc = pltpu.async_copy(cm_hbm.at[meid], crows, semc)
        cpa.wait()
        cpc.wait()
        lax.fori_loop(0, WAVE, accumulate_wave, 0, unroll=2)

    def do_partition(p):
        lo = p * PART

        def init_feat(i, _):
            o = pl.ds(i * L, L)
            acc_s[o] = jnp.zeros((L,), jnp.float32)
            acc_sq[o] = jnp.zeros((L,), jnp.float32)
            acc_mx[o] = jnp.full((L,), _NEG, jnp.float32)
            acc_mn[o] = jnp.full((L,), _POS, jnp.float32)
            return 0

        lax.fori_loop(0, (PART + 1) * D // L, init_feat, 0, unroll=4)

        def init_deg(i, _):
            degv[pl.ds(i * L, L)] = jnp.zeros((L,), jnp.float32)
            return 0

        lax.fori_loop(0, 160 // L, init_deg, 0, unroll=4)

        for w in range(WAVE // L):
            msrc[pl.ds(w * L, L)] = zero16i
            meid[pl.ds(w * L, L)] = zero16i

        pltpu.sync_copy(b_hbm.at[pl.ds(lo, PART)], bpart.at[pl.ds(0, PART)])

        def issue_chunk(c, sbuf, dbuf, sem):
            pltpu.async_copy(src_hbm.at[pl.ds(c * CHUNK, CHUNK)], sbuf, sem)
            pltpu.async_copy(dst_hbm.at[pl.ds(c * CHUNK, CHUNK)], dbuf, sem)

        def wait_chunk(sbuf, dbuf, sem):
            pltpu.make_async_copy(src_hbm.at[pl.ds(0, CHUNK)], sbuf, sem).wait()
            pltpu.make_async_copy(dst_hbm.at[pl.ds(0, CHUNK)], dbuf, sem).wait()

        def scan_chunk(c, sbuf, dbuf, voff_in):
            def vec_iter(v, voff):
                dv = dbuf[pl.ds(v * L, L)]
                sv = sbuf[pl.ds(v * L, L)]
                mask = (dv >= lo) & (dv < lo + PART)
                pc = plsc.cumsum(jnp.where(mask, 1, 0).astype(jnp.int32))
                pos = voff + pc - 1
                dloc = dv - lo
                eidv = c * CHUNK + v * L + iota
                plsc.store_scatter(msrc, [pos], sv, mask=mask)
                plsc.store_scatter(meid, [pos], eidv, mask=mask)
                plsc.store_scatter(mdst, [pos], dloc, mask=mask)
                voff = voff + plsc.all_reduce_population_count(mask)

                def check():
                    n = voff[0]

                    def drain_now():
                        drain(voff)
                        return jnp.zeros((L,), jnp.int32)

                    return lax.cond(n >= DRAIN_AT, drain_now, lambda: voff)

                return lax.cond(v % 2 == 1, check, lambda: voff)

            return lax.fori_loop(0, CHUNK // L, vec_iter, voff_in, unroll=2)

        issue_chunk(0, srcc0, dstc0, semch0)
        issue_chunk(1, srcc1, dstc1, semch1)

        def do_pair(g, voff):
            c0 = 2 * g
            wait_chunk(srcc0, dstc0, semch0)
            voff = scan_chunk(c0, srcc0, dstc0, voff)

            @pl.when(c0 + 2 < NCHUNK)
            def _():
                issue_chunk(c0 + 2, srcc0, dstc0, semch0)

            wait_chunk(srcc1, dstc1, semch1)
            voff = scan_chunk(c0 + 1, srcc1, dstc1, voff)

            @pl.when(c0 + 3 < NCHUNK)
            def _():
                issue_chunk(c0 + 3, srcc1, dstc1, semch1)

            return voff

        voff = lax.fori_loop(0, NCHUNK // 2, do_pair,
                             jnp.zeros((L,), jnp.int32))
        tail = voff[0]
        lax.cond(tail > 0, lambda: drain(voff), lambda: None)

        fo = pl.ds(0, PART * D)
        pltpu.sync_copy(acc_s.at[fo], s_out.at[pl.ds(lo * D, PART * D)])
        pltpu.sync_copy(acc_mx.at[fo], mx_out.at[pl.ds(lo * D, PART * D)])
        pltpu.sync_copy(acc_mn.at[fo], mn_out.at[pl.ds(lo * D, PART * D)])
        pltpu.sync_copy(acc_sq.at[fo], sq_out.at[pl.ds(lo * D, PART * D)])
        pltpu.sync_copy(degv.at[pl.ds(0, PART)], deg_out.at[pl.ds(lo, PART)])

    for k in range((NPART + NW - 1) // NW):
        p = wid + k * NW

        @pl.when(p < NPART)
        def _():
            do_partition(p)


def _sc_aggregate(a, b, cm, src, dst):
    f32 = jnp.float32
    flat = jax.ShapeDtypeStruct((NPAD * D,), f32)
    fn = pl.kernel(
        _sc_body,
        out_type=(flat, flat, flat, flat, jax.ShapeDtypeStruct((NPAD,), f32)),
        mesh=plsc.VectorSubcoreMesh(core_axis_name="c", subcore_axis_name="s",
                                    num_cores=NC, num_subcores=NS),
        compiler_params=pltpu.CompilerParams(needs_layout_passes=False),
        scratch_types=[
            pltpu.VMEM(((PART + 1) * D,), f32),
            pltpu.VMEM(((PART + 1) * D,), f32),
            pltpu.VMEM(((PART + 1) * D,), f32),
            pltpu.VMEM(((PART + 1) * D,), f32),
            pltpu.VMEM((160,), f32),
            pltpu.VMEM((PART + 1, D), f32),
            pltpu.VMEM((CHUNK,), jnp.int32),
            pltpu.VMEM((CHUNK,), jnp.int32),
            pltpu.VMEM((CHUNK,), jnp.int32),
            pltpu.VMEM((CHUNK,), jnp.int32),
            pltpu.VMEM((WAVE,), jnp.int32),
            pltpu.VMEM((WAVE,), jnp.int32),
            pltpu.VMEM((WAVE + L,), jnp.int32),
            pltpu.VMEM((WAVE, D), f32),
            pltpu.VMEM((WAVE, D), f32),
            pltpu.SemaphoreType.DMA,
            pltpu.SemaphoreType.DMA,
            pltpu.SemaphoreType.DMA,
            pltpu.SemaphoreType.DMA,
        ],
    )
    return fn(a, b, cm, src, dst)


def _post_kernel(h_ref, s_ref, mx_ref, mn_ref, sq_ref, deg_ref,
                 w0_ref, w1_ref, w2_ref, w3_ref, b_ref, o_ref):
    deg = deg_ref[...]
    sd = jnp.maximum(deg, 1.0)
    inv = 1.0 / sd
    has_in = deg > 0.0
    mean = s_ref[...] * inv
    mean_sq = sq_ref[...] * inv
    std = jnp.sqrt(jnp.maximum(mean_sq - mean * mean, 0.0) + 1e-30)
    mx = jnp.where(has_in, mx_ref[...], 0.0)
    mn = jnp.where(has_in, mn_ref[...], 0.0)
    agg = jnp.concatenate([mean, mx, mn, std], axis=1)
    logd = jnp.log(deg + 1.0)
    amp = logd * (1.0 / AVG_D_LOG)
    att = jnp.where(has_in, AVG_D_LOG / jnp.where(logd > 0.0, logd, 1.0), 0.0)
    acc = jnp.dot(h_ref[...], w0_ref[...], preferred_element_type=jnp.float32)
    acc += jnp.dot(agg, w1_ref[...], preferred_element_type=jnp.float32)
    acc += amp * jnp.dot(agg, w2_ref[...], preferred_element_type=jnp.float32)
    acc += att * jnp.dot(agg, w3_ref[...], preferred_element_type=jnp.float32)
    o_ref[...] = jnp.maximum(acc + b_ref[...], 0.0)


def _posttrans(hp, s, mx, mn, sq, deg2, w0, w1, w2, w3, b_post):
    bm = 1024
    d4 = 4 * D
    return pl.pallas_call(
        _post_kernel,
        grid=(NPAD // bm,),
        in_specs=[
            pl.BlockSpec((bm, D), lambda i: (i, 0)),
            pl.BlockSpec((bm, D), lambda i: (i, 0)),
            pl.BlockSpec((bm, D), lambda i: (i, 0)),
            pl.BlockSpec((bm, D), lambda i: (i, 0)),
            pl.BlockSpec((bm, D), lambda i: (i, 0)),
            pl.BlockSpec((bm, 1), lambda i: (i, 0)),
            pl.BlockSpec((D, D), lambda i: (0, 0)),
            pl.BlockSpec((d4, D), lambda i: (0, 0)),
            pl.BlockSpec((d4, D), lambda i: (0, 0)),
            pl.BlockSpec((d4, D), lambda i: (0, 0)),
            pl.BlockSpec((D,), lambda i: (0,)),
        ],
        out_specs=pl.BlockSpec((bm, D), lambda i: (i, 0)),
        out_shape=jax.ShapeDtypeStruct((NPAD, D), jnp.float32),
    )(hp, s, mx, mn, sq, deg2, w0, w1, w2, w3, b_post)


def kernel(h, edge_index, e, W_pre, b_pre, W_post, b_post):
    src = edge_index[0]
    dst = edge_index[1]
    w_s = W_pre[:D]
    w_d = W_pre[D:2 * D]
    w_e = W_pre[2 * D:]

    hp = jnp.pad(h, ((0, NPAD - N), (0, 0)))
    a, b = _pre_matmuls(hp, w_s, w_d)
    cm = _matmul(e, w_e, b_pre, block_m=4000)

    s, mx, mn, sq, deg = _sc_aggregate(a, b, cm, src, dst)
    s = s.reshape(NPAD, D)
    mx = mx.reshape(NPAD, D)
    mn = mn.reshape(NPAD, D)
    sq = sq.reshape(NPAD, D)
    deg2 = deg.reshape(NPAD, 1)

    w0 = W_post[:D]
    w1 = W_post[D:D + 4 * D]
    w2 = W_post[D + 4 * D:D + 8 * D]
    w3 = W_post[D + 8 * D:]
    out = _posttrans(hp, s, mx, mn, sq, deg2, w0, w1, w2, w3, b_post)
    return out[:N]


# speedup vs baseline: 2.9467x; 2.9467x over previous
import functools

import jax
import jax.numpy as jnp
from jax import lax
from jax.experimental import pallas as pl
from jax.experimental.pallas import tpu as pltpu
from jax.experimental.pallas import tpu_sc as plsc

N = 10000
E = 320000
D = 128
D_EDGE = 16
AVG_D_LOG = 3.5

NC = 2
NS = 16
L = 16
NW = NC * NS
NPART = 64
PART = 160
NPAD = NPART * PART
CHUNK = 1600
NCHUNK = E // CHUNK
WAVE = 64
R = D // L

_NEG = -1e30
_POS = 1e30


def _mm_kernel(x_ref, w_ref, b_ref, o_ref, *, relu):
    acc = jnp.dot(x_ref[...], w_ref[...], preferred_element_type=jnp.float32)
    acc = acc + b_ref[...]
    if relu:
        acc = jnp.maximum(acc, 0.0)
    o_ref[...] = acc


def _matmul(x, w, b, *, block_m, relu=False):
    m, k = x.shape
    _, n = w.shape
    return pl.pallas_call(
        functools.partial(_mm_kernel, relu=relu),
        grid=(m // block_m,),
        in_specs=[
            pl.BlockSpec((block_m, k), lambda i: (i, 0)),
            pl.BlockSpec((k, n), lambda i: (0, 0)),
            pl.BlockSpec((n,), lambda i: (0,)),
        ],
        out_specs=pl.BlockSpec((block_m, n), lambda i: (i, 0)),
        out_shape=jax.ShapeDtypeStruct((m, n), jnp.float32),
    )(x, w, b)


def _pre_kernel(h_ref, ws_ref, wd_ref, a_ref, b_ref):
    h = h_ref[...]
    a_ref[...] = jnp.dot(h, ws_ref[...], preferred_element_type=jnp.float32)
    b_ref[...] = jnp.dot(h, wd_ref[...], preferred_element_type=jnp.float32)


def _pre_matmuls(hp, w_s, w_d):
    bm = 1024
    return pl.pallas_call(
        _pre_kernel,
        grid=(NPAD // bm,),
        in_specs=[
            pl.BlockSpec((bm, D), lambda i: (i, 0)),
            pl.BlockSpec((D, D), lambda i: (0, 0)),
            pl.BlockSpec((D, D), lambda i: (0, 0)),
        ],
        out_specs=[
            pl.BlockSpec((bm, D), lambda i: (i, 0)),
            pl.BlockSpec((bm, D), lambda i: (i, 0)),
        ],
        out_shape=[
            jax.ShapeDtypeStruct((NPAD, D), jnp.float32),
            jax.ShapeDtypeStruct((NPAD, D), jnp.float32),
        ],
    )(hp, w_s, w_d)


def _sc_body(a_hbm, b_hbm, cm_hbm, src_hbm, dst_hbm,
             s_out, mx_out, mn_out, sq_out, deg_out,
             acc_s, acc_mx, acc_mn, acc_sq, degv, bpart,
             srcc0, dstc0, srcc1, dstc1, msrc, meid, mdst, arows, crows,
             sema, semc, semch0, semch1):
    wid = lax.axis_index("s") * NC + lax.axis_index("c")
    iota = lax.iota(jnp.int32, L)
    ones = jnp.ones((L,), jnp.float32)
    zero16i = jnp.zeros((L,), jnp.int32)

    e0 = jnp.where(iota == 0, 1.0, 0.0).astype(jnp.float32)

    def accumulate_wave(j, _):
        d = mdst[pl.ds(j, L)][0]

        @pl.when(d != PART)
        def _():
            base = d * D
            dg = pl.ds(d, L)
            degv[dg] = degv[dg] + e0
            for r in range(R):
                av = arows[j, pl.ds(r * L, L)]
                cv = crows[j, pl.ds(r * L, L)]
                bv = bpart[d, pl.ds(r * L, L)]
                m = jnp.maximum(av + cv + bv, 0.0)
                o = pl.ds(base + r * L, L)
                plsc.addupdate(acc_s.at[o], m)
                plsc.addupdate(acc_sq.at[o], m * m)
                acc_mx[o] = jnp.maximum(acc_mx[o], m)
                acc_mn[o] = jnp.minimum(acc_mn[o], m)

        return 0

    def pad_and_drain(off):
        for w in range(WAVE // L):
            g = w * L + iota
            pm = g >= off
            o = pl.ds(w * L, L)
            msrc[o] = jnp.where(pm, zero16i, msrc[o])
            meid[o] = jnp.where(pm, zero16i, meid[o])
            mdst[o] = jnp.where(pm, jnp.full((L,), PART, jnp.int32), mdst[o])
        cpa = pltpu.async_copy(a_hbm.at[msrc], arows, sema)
        cpc = pltpu.async_copy(cm_hbm.at[meid], crows, semc)
        cpa.wait()
        cpc.wait()
        lax.fori_loop(0, WAVE, accumulate_wave, 0, unroll=2)

    def do_partition(p):
        lo = p * PART

        def init_feat(i, _):
            o = pl.ds(i * L, L)
            acc_s[o] = jnp.zeros((L,), jnp.float32)
            acc_sq[o] = jnp.zeros((L,), jnp.float32)
            acc_mx[o] = jnp.full((L,), _NEG, jnp.float32)
            acc_mn[o] = jnp.full((L,), _POS, jnp.float32)
            return 0

        lax.fori_loop(0, (PART + 1) * D // L, init_feat, 0, unroll=4)

        def init_deg(i, _):
            degv[pl.ds(i * L, L)] = jnp.zeros((L,), jnp.float32)
            return 0

        lax.fori_loop(0, 192 // L, init_deg, 0, unroll=4)

        pltpu.sync_copy(b_hbm.at[pl.ds(lo, PART)], bpart.at[pl.ds(0, PART)])

        def issue_chunk(c, sbuf, dbuf, sem):
            pltpu.async_copy(src_hbm.at[pl.ds(c * CHUNK, CHUNK)], sbuf, sem)
            pltpu.async_copy(dst_hbm.at[pl.ds(c * CHUNK, CHUNK)], dbuf, sem)

        def wait_chunk(sbuf, dbuf, sem):
            pltpu.make_async_copy(src_hbm.at[pl.ds(0, CHUNK)], sbuf, sem).wait()
            pltpu.make_async_copy(dst_hbm.at[pl.ds(0, CHUNK)], dbuf, sem).wait()

        def scan_chunk(c, sbuf, dbuf, off_in):
            def vec_iter(v, off):
                dv = dbuf[pl.ds(v * L, L)]
                mask = (dv >= lo) & (dv < lo + PART)
                cnt = plsc.all_reduce_population_count(mask)[0]

                def matched():
                    sv = sbuf[pl.ds(v * L, L)]
                    dloc = dv - lo
                    eidv = c * CHUNK + v * L + iota
                    plsc.store_compressed(msrc.at[pl.ds(off, L)], sv,
                                          mask=mask)
                    plsc.store_compressed(meid.at[pl.ds(off, L)], eidv,
                                          mask=mask)
                    plsc.store_compressed(mdst.at[pl.ds(off, L)], dloc,
                                          mask=mask)
                    new_off = off + cnt

                    def drain_now():
                        pad_and_drain(new_off)
                        return jnp.int32(0)

                    return lax.cond(new_off >= WAVE - (L - 1), drain_now,
                                    lambda: new_off)

                return lax.cond(cnt > 0, matched, lambda: off)

            return lax.fori_loop(0, CHUNK // L, vec_iter, off_in, unroll=2)

        issue_chunk(0, srcc0, dstc0, semch0)
        issue_chunk(1, srcc1, dstc1, semch1)

        def do_pair(g, off):
            c0 = 2 * g
            wait_chunk(srcc0, dstc0, semch0)
            off = scan_chunk(c0, srcc0, dstc0, off)

            @pl.when(c0 + 2 < NCHUNK)
            def _():
                issue_chunk(c0 + 2, srcc0, dstc0, semch0)

            wait_chunk(srcc1, dstc1, semch1)
            off = scan_chunk(c0 + 1, srcc1, dstc1, off)

            @pl.when(c0 + 3 < NCHUNK)
            def _():
                issue_chunk(c0 + 3, srcc1, dstc1, semch1)

            return off

        off = lax.fori_loop(0, NCHUNK // 2, do_pair, jnp.int32(0))
        lax.cond(off > 0, lambda: pad_and_drain(off), lambda: None)

        fo = pl.ds(0, PART * D)
        pltpu.sync_copy(acc_s.at[fo], s_out.at[pl.ds(lo * D, PART * D)])
        pltpu.sync_copy(acc_mx.at[fo], mx_out.at[pl.ds(lo * D, PART * D)])
        pltpu.sync_copy(acc_mn.at[fo], mn_out.at[pl.ds(lo * D, PART * D)])
        pltpu.sync_copy(acc_sq.at[fo], sq_out.at[pl.ds(lo * D, PART * D)])
        pltpu.sync_copy(degv.at[pl.ds(0, PART)], deg_out.at[pl.ds(lo, PART)])

    for k in range(NPART // NW):
        do_partition(wid + k * NW)


def _sc_aggregate(a, b, cm, src, dst):
    f32 = jnp.float32
    flat = jax.ShapeDtypeStruct((NPAD * D,), f32)
    fn = pl.kernel(
        _sc_body,
        out_type=(flat, flat, flat, flat, jax.ShapeDtypeStruct((NPAD,), f32)),
        mesh=plsc.VectorSubcoreMesh(core_axis_name="c", subcore_axis_name="s",
                                    num_cores=NC, num_subcores=NS),
        compiler_params=pltpu.CompilerParams(needs_layout_passes=False),
        scratch_types=[
            pltpu.VMEM(((PART + 1) * D,), f32),
            pltpu.VMEM(((PART + 1) * D,), f32),
            pltpu.VMEM(((PART + 1) * D,), f32),
            pltpu.VMEM(((PART + 1) * D,), f32),
            pltpu.VMEM((192,), f32),
            pltpu.VMEM((PART + 1, D), f32),
            pltpu.VMEM((CHUNK,), jnp.int32),
            pltpu.VMEM((CHUNK,), jnp.int32),
            pltpu.VMEM((CHUNK,), jnp.int32),
            pltpu.VMEM((CHUNK,), jnp.int32),
            pltpu.VMEM((WAVE,), jnp.int32),
            pltpu.VMEM((WAVE,), jnp.int32),
            pltpu.VMEM((WAVE + L,), jnp.int32),
            pltpu.VMEM((WAVE, D), f32),
            pltpu.VMEM((WAVE, D), f32),
            pltpu.SemaphoreType.DMA,
            pltpu.SemaphoreType.DMA,
            pltpu.SemaphoreType.DMA,
            pltpu.SemaphoreType.DMA,
        ],
    )
    return fn(a, b, cm, src, dst)


def _post_kernel(h_ref, s_ref, mx_ref, mn_ref, sq_ref, deg_ref,
                 w0_ref, w1_ref, w2_ref, w3_ref, b_ref, o_ref):
    deg = deg_ref[...]
    sd = jnp.maximum(deg, 1.0)
    inv = 1.0 / sd
    has_in = deg > 0.0
    mean = s_ref[...] * inv
    mean_sq = sq_ref[...] * inv
    std = jnp.sqrt(jnp.maximum(mean_sq - mean * mean, 0.0) + 1e-30)
    mx = jnp.where(has_in, mx_ref[...], 0.0)
    mn = jnp.where(has_in, mn_ref[...], 0.0)
    agg = jnp.concatenate([mean, mx, mn, std], axis=1)
    logd = jnp.log(deg + 1.0)
    amp = logd * (1.0 / AVG_D_LOG)
    att = jnp.where(has_in, AVG_D_LOG / jnp.where(logd > 0.0, logd, 1.0), 0.0)
    acc = jnp.dot(h_ref[...], w0_ref[...], preferred_element_type=jnp.float32)
    acc += jnp.dot(agg, w1_ref[...], preferred_element_type=jnp.float32)
    acc += amp * jnp.dot(agg, w2_ref[...], preferred_element_type=jnp.float32)
    acc += att * jnp.dot(agg, w3_ref[...], preferred_element_type=jnp.float32)
    o_ref[...] = jnp.maximum(acc + b_ref[...], 0.0)


def _posttrans(hp, s, mx, mn, sq, deg2, w0, w1, w2, w3, b_post):
    bm = 1024
    d4 = 4 * D
    return pl.pallas_call(
        _post_kernel,
        grid=(NPAD // bm,),
        in_specs=[
            pl.BlockSpec((bm, D), lambda i: (i, 0)),
            pl.BlockSpec((bm, D), lambda i: (i, 0)),
            pl.BlockSpec((bm, D), lambda i: (i, 0)),
            pl.BlockSpec((bm, D), lambda i: (i, 0)),
            pl.BlockSpec((bm, D), lambda i: (i, 0)),
            pl.BlockSpec((bm, 1), lambda i: (i, 0)),
            pl.BlockSpec((D, D), lambda i: (0, 0)),
            pl.BlockSpec((d4, D), lambda i: (0, 0)),
            pl.BlockSpec((d4, D), lambda i: (0, 0)),
            pl.BlockSpec((d4, D), lambda i: (0, 0)),
            pl.BlockSpec((D,), lambda i: (0,)),
        ],
        out_specs=pl.BlockSpec((bm, D), lambda i: (i, 0)),
        out_shape=jax.ShapeDtypeStruct((NPAD, D), jnp.float32),
    )(hp, s, mx, mn, sq, deg2, w0, w1, w2, w3, b_post)


def kernel(h, edge_index, e, W_pre, b_pre, W_post, b_post):
    src = edge_index[0]
    dst = edge_index[1]
    w_s = W_pre[:D]
    w_d = W_pre[D:2 * D]
    w_e = W_pre[2 * D:]

    hp = jnp.pad(h, ((0, NPAD - N), (0, 0)))
    a, b = _pre_matmuls(hp, w_s, w_d)
    cm = _matmul(e, w_e, b_pre, block_m=4000)

    s, mx, mn, sq, deg = _sc_aggregate(a, b, cm, src, dst)
    s = s.reshape(NPAD, D)
    mx = mx.reshape(NPAD, D)
    mn = mn.reshape(NPAD, D)
    sq = sq.reshape(NPAD, D)
    deg2 = deg.reshape(NPAD, 1)

    w0 = W_post[:D]
    w1 = W_post[D:D + 4 * D]
    w2 = W_post[D + 4 * D:D + 8 * D]
    w3 = W_post[D + 8 * D:]
    out = _posttrans(hp, s, mx, mn, sq, deg2, w0, w1, w2, w3, b_post)
    return out[:N]
